# baseline (speedup 1.0000x reference)
import functools

import jax
import jax.numpy as jnp
from jax import lax
from jax.experimental import pallas as pl
from jax.experimental.pallas import tpu as pltpu
from jax.experimental.pallas import tpu_sc as plsc

N_NODES = 10000
D_FEAT = 128
N_EDGES = 320000

NC = 2
NS = 16
NW = NC * NS

CHUNK = 128
ACC_ROWS = 10112
ROWS_PER_TILE = ACC_ROWS // NS

T_RAW = -(-N_EDGES // (NW * CHUNK))
T_CHUNKS = T_RAW + (T_RAW % 2)
E_PAD = NW * CHUNK * T_CHUNKS


def _sc_partial_sums(x, src, dst, zeros):
  mesh = plsc.VectorSubcoreMesh(core_axis_name="c", subcore_axis_name="s")

  @functools.partial(
      pl.kernel,
      mesh=mesh,
      out_type=jax.ShapeDtypeStruct((NC, ACC_ROWS, D_FEAT), jnp.float32),
      scratch_types=[
          pltpu.VMEM((T_CHUNKS, CHUNK), jnp.int32),
          pltpu.VMEM((CHUNK,), jnp.int32),
          pltpu.VMEM((CHUNK,), jnp.int32),
          pltpu.VMEM((CHUNK, D_FEAT), jnp.float32),
          pltpu.VMEM((CHUNK, D_FEAT), jnp.float32),
          pltpu.VMEM_SHARED((ACC_ROWS, D_FEAT), jnp.float32),
          pltpu.SemaphoreType.DMA,
          pltpu.SemaphoreType.DMA,
          pltpu.SemaphoreType.DMA,
          pltpu.SemaphoreType.DMA,
      ],
  )
  def k(x_hbm, src_hbm, dst_hbm, zeros_hbm, out_hbm,
        dst_all, sidx0, sidx1, buf0, buf1, acc, gsem0, gsem1, isem0, isem1):
    c = lax.axis_index("c")
    s = lax.axis_index("s")
    wid = s * NC + c

    pltpu.sync_copy(zeros_hbm, acc.at[pl.ds(s * ROWS_PER_TILE, ROWS_PER_TILE)])
    pltpu.sync_copy(dst_hbm.at[wid], dst_all)
    plsc.subcore_barrier()

    def idx_load(t, sidx, isem):
      return pltpu.make_async_copy(
          src_hbm.at[pl.ds((wid * T_CHUNKS + t) * CHUNK, CHUNK)], sidx, isem)

    def gather(sidx, buf, gsem):
      return pltpu.make_async_copy(x_hbm.at[sidx], buf, gsem)

    def scatter_add(t, buf):
      pltpu.sync_copy(buf, acc.at[dst_all.at[t]], add=True)

    idx_load(0, sidx0, isem0).start()
    idx_load(0, sidx0, isem0).wait()
    gather(sidx0, buf0, gsem0).start()
    idx_load(1, sidx1, isem1).start()

    def body(i, carry):
      t0 = 2 * i
      t1 = 2 * i + 1
      gather(sidx0, buf0, gsem0).wait()

      @pl.when(i < T_CHUNKS // 2 - 1)
      def _():
        idx_load(t0 + 2, sidx0, isem0).start()

      idx_load(t1, sidx1, isem1).wait()
      gather(sidx1, buf1, gsem1).start()
      scatter_add(t0, buf0)

      @pl.when(i < T_CHUNKS // 2 - 1)
      def _():
        idx_load(t0 + 2, sidx0, isem0).wait()
        gather(sidx0, buf0, gsem0).start()
        idx_load(t1 + 2, sidx1, isem1).start()

      gather(sidx1, buf1, gsem1).wait()
      scatter_add(t1, buf1)
      return carry

    lax.fori_loop(0, T_CHUNKS // 2, body, 0)
    plsc.subcore_barrier()

    pltpu.sync_copy(
        acc.at[pl.ds(s * ROWS_PER_TILE, ROWS_PER_TILE)],
        out_hbm.at[c, pl.ds(s * ROWS_PER_TILE, ROWS_PER_TILE)],
    )

  return k(x, src, dst, zeros)


def _combine_body(a_ref, b_ref, o_ref):
  o_ref[...] = a_ref[0] + b_ref[0]


_BLK = 1000


def _combine(partials):
  return pl.pallas_call(
      _combine_body,
      grid=(N_NODES // _BLK,),
      in_specs=[
          pl.BlockSpec((1, _BLK, D_FEAT), lambda i: (0, i, 0)),
          pl.BlockSpec((1, _BLK, D_FEAT), lambda i: (1, i, 0)),
      ],
      out_specs=pl.BlockSpec((_BLK, D_FEAT), lambda i: (i, 0)),
      out_shape=jax.ShapeDtypeStruct((N_NODES, D_FEAT), jnp.float32),
  )(partials, partials)


def kernel(X, edge_index):
  pad = E_PAD - N_EDGES
  src = jnp.concatenate([edge_index[1], jnp.zeros((pad,), jnp.int32)])
  dst = jnp.concatenate(
      [edge_index[0], jnp.full((pad,), N_NODES, jnp.int32)]).reshape(NW, T_CHUNKS, CHUNK)
  zeros = jnp.zeros((ROWS_PER_TILE, D_FEAT), jnp.float32)
  partials = _sc_partial_sums(X, src, dst, zeros)
  return _combine(partials)


# speedup vs baseline: 3.4823x; 3.4823x over previous
import functools

import jax
import jax.numpy as jnp
from jax import lax
from jax.experimental import pallas as pl
from jax.experimental.pallas import tpu as pltpu
from jax.experimental.pallas import tpu_sc as plsc

N_NODES = 10000
D_FEAT = 128
N_EDGES = 320000

NC = 2
NS = 16
NW = NC * NS

CHUNK = 128
ACC_ROWS = 10112
ROWS_PER_TILE = ACC_ROWS // NS

T_RAW = -(-N_EDGES // (NW * CHUNK))
T_CHUNKS = T_RAW + (T_RAW % 2)
E_PAD = NW * CHUNK * T_CHUNKS


def _sc_partial_sums(x, src, dst, zeros):
  mesh = plsc.VectorSubcoreMesh(core_axis_name="c", subcore_axis_name="s")

  @functools.partial(
      pl.kernel,
      mesh=mesh,
      out_type=jax.ShapeDtypeStruct((NC, ACC_ROWS, D_FEAT), jnp.float32),
      scratch_types=[
          pltpu.VMEM((T_CHUNKS, CHUNK), jnp.int32),
          pltpu.VMEM((CHUNK,), jnp.int32),
          pltpu.VMEM((CHUNK,), jnp.int32),
          pltpu.VMEM((CHUNK, D_FEAT), jnp.float32),
          pltpu.VMEM((CHUNK, D_FEAT), jnp.float32),
          pltpu.VMEM_SHARED((ACC_ROWS, D_FEAT), jnp.float32),
          pltpu.SemaphoreType.DMA,
          pltpu.SemaphoreType.DMA,
          pltpu.SemaphoreType.DMA,
          pltpu.SemaphoreType.DMA,
      ],
  )
  def k(x_hbm, src_hbm, dst_hbm, zeros_hbm, out_hbm,
        dst_all, sidx0, sidx1, buf0, buf1, acc, gsem0, gsem1, isem0, isem1):
    c = lax.axis_index("c")
    s = lax.axis_index("s")
    wid = s * NC + c

    pltpu.sync_copy(zeros_hbm, acc.at[pl.ds(s * ROWS_PER_TILE, ROWS_PER_TILE)])
    pltpu.sync_copy(dst_hbm.at[wid], dst_all)
    plsc.subcore_barrier()

    def idx_load(t, sidx, isem):
      return pltpu.make_async_copy(
          src_hbm.at[pl.ds((wid * T_CHUNKS + t) * CHUNK, CHUNK)], sidx, isem)

    def gather(sidx, buf, gsem):
      return pltpu.make_async_copy(x_hbm.at[sidx], buf, gsem)

    def scatter_add(t, buf):
      pltpu.sync_copy(buf, acc.at[dst_all.at[t]], add=True)

    idx_load(0, sidx0, isem0).start()
    idx_load(0, sidx0, isem0).wait()
    gather(sidx0, buf0, gsem0).start()
    idx_load(1, sidx1, isem1).start()

    def body(i, carry):
      t0 = 2 * i
      t1 = 2 * i + 1
      gather(sidx0, buf0, gsem0).wait()

      @pl.when(i < T_CHUNKS // 2 - 1)
      def _():
        idx_load(t0 + 2, sidx0, isem0).start()

      idx_load(t1, sidx1, isem1).wait()
      gather(sidx1, buf1, gsem1).start()
      scatter_add(t0, buf0)

      @pl.when(i < T_CHUNKS // 2 - 1)
      def _():
        idx_load(t0 + 2, sidx0, isem0).wait()
        gather(sidx0, buf0, gsem0).start()
        idx_load(t1 + 2, sidx1, isem1).start()

      gather(sidx1, buf1, gsem1).wait()
      scatter_add(t1, buf1)
      return carry

    lax.fori_loop(0, T_CHUNKS // 2, body, 0)
    plsc.subcore_barrier()

    pltpu.sync_copy(
        acc.at[pl.ds(s * ROWS_PER_TILE, ROWS_PER_TILE)],
        out_hbm.at[c, pl.ds(s * ROWS_PER_TILE, ROWS_PER_TILE)],
    )

  return k(x, src, dst, zeros)


def _combine_body(a_ref, b_ref, o_ref):
  o_ref[...] = a_ref[0] + b_ref[0]


_BLK = 1000


def _combine(partials):
  return pl.pallas_call(
      _combine_body,
      grid=(N_NODES // _BLK,),
      in_specs=[
          pl.BlockSpec((1, _BLK, D_FEAT), lambda i: (0, i, 0)),
          pl.BlockSpec((1, _BLK, D_FEAT), lambda i: (1, i, 0)),
      ],
      out_specs=pl.BlockSpec((_BLK, D_FEAT), lambda i: (i, 0)),
      out_shape=jax.ShapeDtypeStruct((N_NODES, D_FEAT), jnp.float32),
  )(partials, partials)


def kernel(X, edge_index):
  pad = E_PAD - N_EDGES
  pad_iota = jnp.arange(pad, dtype=jnp.int32)
  src = jnp.concatenate([edge_index[1], pad_iota % N_NODES])
  dst = jnp.concatenate(
      [edge_index[0],
       N_NODES + pad_iota % (ACC_ROWS - N_NODES)]).reshape(NW, T_CHUNKS, CHUNK)
  zeros = jnp.zeros((ROWS_PER_TILE, D_FEAT), jnp.float32)
  partials = _sc_partial_sums(X, src, dst, zeros)
  return _combine(partials)
